# baseline (speedup 1.0000x reference)
import functools

import jax
import jax.numpy as jnp
import numpy as np
from jax import lax
from jax.experimental import pallas as pl
from jax.experimental.pallas import tpu as pltpu
from jax.experimental.pallas import tpu_sc as plsc

N = 10000
E = 320000
R = 8
D = 128
NP = 10240
AR = NP // D
NC = 2
NS = 16
NWORK = NC * NS
EPW = E // NWORK
CH = 80
NCHUNK = EPW // CH
ROWS_PER_TILE = NP // NS
ARPT = 8
ATILES = AR // ARPT
FB = 1024
NA = 48
NB = CH - NA

def _u_body(emb_ref, rel_ref, wr_ref, u_ref):
    w = wr_ref[0]
    p = jnp.dot(emb_ref[...], w, preferred_element_type=jnp.float32)
    t = jnp.tanh(p + rel_ref[0])
    u_ref[...] = lax.dot_general(
        t, w, (((1,), (1,)), ((), ())),
        preferred_element_type=jnp.float32)


def _precompute_u(emb, rel_embed, W_R):
    B = 1000
    grid = (R, N // B)
    return pl.pallas_call(
        _u_body,
        grid=grid,
        in_specs=[
            pl.BlockSpec((B, D), lambda r, i: (i, 0)),
            pl.BlockSpec((1, 1, D), lambda r, i: (r, 0, 0)),
            pl.BlockSpec((1, D, D), lambda r, i: (r, 0, 0)),
        ],
        out_specs=pl.BlockSpec((B, D), lambda r, i: (r * (N // B) + i, 0)),
        out_shape=jax.ShapeDtypeStruct((R * N, D), jnp.float32),
    )(emb, rel_embed.reshape(R, 1, D), W_R)


def _sc_body(emb_hbm, u_hbm, src_hbm, dst_hbm, et_hbm, nh_out, att_out,
             src_v, dst_v, et_v, gidx_v, arow_v, dstS_v,
             laneA_v, laneB_v, arowA_v, arowB_v,
             erows, urows, miniA, miniB, nh_sh, att_sh,
             sem_ge, sem_gu, sem_snh, sem_sa, sem_sb):
    c = lax.axis_index("c")
    s = lax.axis_index("s")
    wid = s * NC + c
    zero16 = jnp.zeros((16,), jnp.float32)
    lane = lax.iota(jnp.int32, 16)

    def zero_er(i, _):
        for q in range(D // 16):
            erows[0][i, pl.ds(q * 16, 16)] = zero16
        return 0
    lax.fori_loop(0, CH, zero_er, 0)

    def zero_minis(i, _):
        for q in range(D // 16):
            miniA[i, pl.ds(q * 16, 16)] = zero16
        return 0
    lax.fori_loop(0, NA, zero_minis, 0)

    def zero_minib(i, _):
        for q in range(D // 16):
            miniB[i, pl.ds(q * 16, 16)] = zero16
        return 0
    lax.fori_loop(0, NB, zero_minib, 0)

    for j in range(ROWS_PER_TILE // CH):
        pltpu.sync_copy(erows[0],
                        nh_sh.at[pl.ds(s * ROWS_PER_TILE + j * CH, CH)])

    @pl.when(s < ATILES)
    def _():
        pltpu.sync_copy(erows[0].at[pl.ds(0, ARPT)],
                        att_sh.at[pl.ds(s * ARPT, ARPT)])
    plsc.subcore_barrier()

    def idxload(j, b):
        base = wid * EPW + j * CH
        pltpu.sync_copy(src_hbm.at[pl.ds(base, CH)], src_v[b])
        pltpu.sync_copy(dst_hbm.at[pl.ds(base, CH)], dst_v[b])
        pltpu.sync_copy(et_hbm.at[pl.ds(base, CH)], et_v[b])
        for k in range(CH // 16):
            d16 = dst_v[b][pl.ds(k * 16, 16)]
            gidx_v[b][pl.ds(k * 16, 16)] = (
                et_v[b][pl.ds(k * 16, 16)] * N + d16)
            arow_v[b][pl.ds(k * 16, 16)] = lax.shift_right_logical(d16, 7)

    def egather_start(b):
        pltpu.async_copy(emb_hbm.at[src_v[b]], erows[b], sem_ge[b])

    def ugather_start(b):
        pltpu.async_copy(u_hbm.at[gidx_v[b]], urows, sem_gu)

    def gather_wait(b):
        pltpu.make_async_copy(emb_hbm.at[src_v[b]], erows[b],
                              sem_ge[b]).wait()
        pltpu.make_async_copy(u_hbm.at[gidx_v[b]], urows, sem_gu).wait()

    def dot_edge(b, e):
        eq = tuple(erows[b][e, pl.ds(q * 16, 16)] for q in range(D // 16))
        acc = jnp.zeros((16,), jnp.float32)
        for q in range(D // 16):
            acc = acc + eq[q] * urows[e, pl.ds(q * 16, 16)]
        ae = jnp.exp(jnp.broadcast_to(jnp.sum(acc), (16,)))
        return eq, ae

    def scale_edge(b, e, eq, ae):
        for q in range(D // 16):
            erows[b][e, pl.ds(q * 16, 16)] = eq[q] * ae

    def groups_range(b, lo, hi, mini, off, laneS, arowX):
        def gbody(k, _):
            base_e = k * 16
            eq0, ae0 = dot_edge(b, base_e)

            def ebody(jj, carry):
                ae16, ae_p = carry[0], carry[1]
                eq_p = carry[2:]
                e = base_e + jj
                eqn, aen = dot_edge(b, e)
                scale_edge(b, e - 1, eq_p, ae_p)
                ae16 = jnp.where(lane == (jj - 1), ae_p, ae16)
                return (ae16, aen) + eqn

            carry = lax.fori_loop(1, 16, ebody, (zero16, ae0) + eq0,
                                  unroll=3)
            ae16, ae_l = carry[0], carry[1]
            eq_l = carry[2:]
            scale_edge(b, base_e + 15, eq_l, ae_l)
            ae16 = jnp.where(lane == 15, ae_l, ae16)
            rows_local = lane + (base_e - off)
            d16 = dst_v[b][pl.ds(base_e, 16)]
            dstS_v[b][pl.ds(base_e, 16)] = d16
            lane16 = jnp.bitwise_and(d16, 127)
            plsc.store_scatter(mini, [rows_local, lane16], ae16)
            laneS[pl.ds(base_e - off, 16)] = lane16
            arowX[pl.ds(base_e - off, 16)] = arow_v[b][pl.ds(base_e, 16)]
            return 0
        lax.fori_loop(lo, hi, gbody, 0)

    def clear_mini(mini, nrows, laneS):
        for k in range(nrows // 16):
            rows_local = lane + k * 16
            lane16 = laneS[pl.ds(k * 16, 16)]
            plsc.store_scatter(mini, [rows_local, lane16], zero16)

    def substep(j, b, first, issue_next, issue_u):
        gather_wait(b)
        if not first:
            pltpu.make_async_copy(miniA, att_sh.at[arowA_v], sem_sa).wait()
            clear_mini(miniA, NA, laneA_v)
        groups_range(b, 0, 3, miniA, 0, laneA_v, arowA_v)
        pltpu.async_copy(miniA, att_sh.at[arowA_v], sem_sa, add=True)
        if not first:
            pltpu.make_async_copy(miniB, att_sh.at[arowB_v], sem_sb).wait()
            clear_mini(miniB, NB, laneB_v)
        groups_range(b, 3, CH // 16, miniB, NA, laneB_v, arowB_v)
        pltpu.async_copy(miniB, att_sh.at[arowB_v], sem_sb, add=True)
        if issue_u:
            ugather_start(1 - b)
        pltpu.async_copy(erows[b], nh_sh.at[dstS_v[b]], sem_snh[b], add=True)
        if issue_next:
            idxload(j + 2, b)
        pltpu.make_async_copy(erows[b], nh_sh.at[dstS_v[b]],
                              sem_snh[b]).wait()
        if issue_next:
            egather_start(b)

    idxload(0, 0)
    egather_start(0)
    ugather_start(0)
    idxload(1, 1)
    egather_start(1)
    substep(0, 0, True, True, True)
    substep(1, 1, False, True, True)

    def pair_body(i, _):
        substep(2 * i, 0, False, True, True)
        substep(2 * i + 1, 1, False, True, True)
        return 0
    lax.fori_loop(1, (NCHUNK - 3) // 2, pair_body, 0)

    substep(NCHUNK - 3, 0, False, True, True)
    substep(NCHUNK - 2, 1, False, False, True)
    substep(NCHUNK - 1, 0, False, False, False)
    pltpu.make_async_copy(miniA, att_sh.at[arowA_v], sem_sa).wait()
    pltpu.make_async_copy(miniB, att_sh.at[arowB_v], sem_sb).wait()

    plsc.subcore_barrier()
    pltpu.sync_copy(nh_sh.at[pl.ds(s * ROWS_PER_TILE, ROWS_PER_TILE)],
                    nh_out.at[c, pl.ds(s * ROWS_PER_TILE, ROWS_PER_TILE)])

    @pl.when(s < ATILES)
    def _():
        pltpu.sync_copy(att_sh.at[pl.ds(s * ARPT, ARPT)],
                        att_out.at[c, pl.ds(s * ARPT, ARPT)])


@functools.partial(
    pl.kernel,
    out_type=[jax.ShapeDtypeStruct((NC, NP, D), jnp.float32),
              jax.ShapeDtypeStruct((NC, AR, D), jnp.float32)],
    mesh=plsc.VectorSubcoreMesh(core_axis_name="c", subcore_axis_name="s"),
    compiler_params=pltpu.CompilerParams(needs_layout_passes=False),
    scratch_types=[
        [pltpu.VMEM((CH,), jnp.int32)] * 2,
        [pltpu.VMEM((CH,), jnp.int32)] * 2,
        [pltpu.VMEM((CH,), jnp.int32)] * 2,
        [pltpu.VMEM((CH,), jnp.int32)] * 2,
        [pltpu.VMEM((CH,), jnp.int32)] * 2,
        [pltpu.VMEM((CH,), jnp.int32)] * 2,
        pltpu.VMEM((NA,), jnp.int32),
        pltpu.VMEM((NB,), jnp.int32),
        pltpu.VMEM((NA,), jnp.int32),
        pltpu.VMEM((NB,), jnp.int32),
        [pltpu.VMEM((CH, D), jnp.float32)] * 2,
        pltpu.VMEM((CH, D), jnp.float32),
        pltpu.VMEM((NA, D), jnp.float32),
        pltpu.VMEM((NB, D), jnp.float32),
        pltpu.VMEM_SHARED((NP, D), jnp.float32),
        pltpu.VMEM_SHARED((AR, D), jnp.float32),
        [pltpu.SemaphoreType.DMA] * 2,
        pltpu.SemaphoreType.DMA,
        [pltpu.SemaphoreType.DMA] * 2,
        pltpu.SemaphoreType.DMA,
        pltpu.SemaphoreType.DMA,
    ],
)
def _sc_edge_pass(emb_hbm, u_hbm, src_hbm, dst_hbm, et_hbm, nh_out, att_out,
                  src_v, dst_v, et_v, gidx_v, arow_v, dstS_v,
                  laneA_v, laneB_v, arowA_v, arowB_v,
                  erows, urows, miniA, miniB, nh_sh, att_sh,
                  sem_ge, sem_gu, sem_snh, sem_sa, sem_sb):
    _sc_body(emb_hbm, u_hbm, src_hbm, dst_hbm, et_hbm, nh_out, att_out,
             src_v, dst_v, et_v, gidx_v, arow_v, dstS_v,
             laneA_v, laneB_v, arowA_v, arowB_v,
             erows, urows, miniA, miniB, nh_sh, att_sh,
             sem_ge, sem_gu, sem_snh, sem_sa, sem_sb)


def _fin_body(emb_ref, nh_ref, att_ref, w1t_ref, b1_ref, w2t_ref, b2_ref,
              o_ref):
    att_sum = att_ref[0] + att_ref[1]
    nh = (nh_ref[0] + nh_ref[1]) / (att_sum + 1e-10)
    e = emb_ref[...]
    h1 = jnp.dot(e + nh, w1t_ref[...],
                 preferred_element_type=jnp.float32) + b1_ref[...]
    h2 = jnp.dot(e * nh, w2t_ref[...],
                 preferred_element_type=jnp.float32) + b2_ref[...]
    h1 = jnp.where(h1 > 0, h1, 0.01 * h1)
    h2 = jnp.where(h2 > 0, h2, 0.01 * h2)
    o = h1 + h2
    nrm = jnp.sqrt(jnp.sum(o * o, axis=1, keepdims=True))
    o_ref[...] = o / jnp.maximum(nrm, 1e-12)


def _finalize(emb_p, nh, att, W1t, b1, W2t, b2):
    OUT = W1t.shape[1]
    return pl.pallas_call(
        _fin_body,
        grid=(NP // FB,),
        in_specs=[
            pl.BlockSpec((FB, D), lambda i: (i, 0)),
            pl.BlockSpec((NC, FB, D), lambda i: (0, i, 0)),
            pl.BlockSpec((NC, FB, 1), lambda i: (0, i, 0)),
            pl.BlockSpec((D, OUT), lambda i: (0, 0)),
            pl.BlockSpec((1, OUT), lambda i: (0, 0)),
            pl.BlockSpec((D, OUT), lambda i: (0, 0)),
            pl.BlockSpec((1, OUT), lambda i: (0, 0)),
        ],
        out_specs=pl.BlockSpec((FB, OUT), lambda i: (i, 0)),
        out_shape=jax.ShapeDtypeStruct((NP, OUT), jnp.float32),
    )(emb_p, nh, att, W1t, b1, W2t, b2)


def kernel(node_ids, edge_index, edge_type, emb_table, rel_embed, W_R,
           W1, b1, W2, b2):
    emb = jnp.take(emb_table, node_ids, axis=0)
    u = _precompute_u(emb, rel_embed, W_R)
    nh, att = _sc_edge_pass(emb, u, edge_index[0], edge_index[1], edge_type)
    att_r = att.reshape(NC, NP, 1)
    emb_p = jnp.concatenate(
        [emb, jnp.zeros((NP - N, D), jnp.float32)], axis=0)
    norm_embed = _finalize(emb_p, nh, att_r, W1.T, b1.reshape(1, -1),
                           W2.T, b2.reshape(1, -1))[:N]
    return jnp.concatenate([emb, norm_embed], axis=1)


# speedup vs baseline: 1.0553x; 1.0553x over previous
import functools

import jax
import jax.numpy as jnp
import numpy as np
from jax import lax
from jax.experimental import pallas as pl
from jax.experimental.pallas import tpu as pltpu
from jax.experimental.pallas import tpu_sc as plsc

N = 10000
E = 320000
R = 8
D = 128
NP = 10240
AR = NP // D
NC = 2
NS = 16
NWORK = NC * NS
EPW = E // NWORK
CH = 80
NCHUNK = EPW // CH
ROWS_PER_TILE = NP // NS
ARPT = 8
ATILES = AR // ARPT
FB = 1024
NA = 48
NB = CH - NA

def _u_body(emb_ref, rel_ref, wr_ref, u_ref):
    w = wr_ref[0]
    p = jnp.dot(emb_ref[...], w, preferred_element_type=jnp.float32)
    t = jnp.tanh(p + rel_ref[0])
    u_ref[...] = lax.dot_general(
        t, w, (((1,), (1,)), ((), ())),
        preferred_element_type=jnp.float32)


def _precompute_u(emb, rel_embed, W_R):
    B = 1000
    grid = (R, N // B)
    return pl.pallas_call(
        _u_body,
        grid=grid,
        in_specs=[
            pl.BlockSpec((B, D), lambda r, i: (i, 0)),
            pl.BlockSpec((1, 1, D), lambda r, i: (r, 0, 0)),
            pl.BlockSpec((1, D, D), lambda r, i: (r, 0, 0)),
        ],
        out_specs=pl.BlockSpec((B, D), lambda r, i: (r * (N // B) + i, 0)),
        out_shape=jax.ShapeDtypeStruct((R * N, D), jnp.float32),
    )(emb, rel_embed.reshape(R, 1, D), W_R)


def _sc_body(emb_hbm, u_hbm, sde_hbm, nh_out, att_out,
             sde_v, gidx_v, arow_v, dstS_v,
             laneA_v, laneB_v, arowA_v, arowB_v,
             erows, urows, miniA, miniB, nh_sh, att_sh,
             sem_ge, sem_gu, sem_snh, sem_idx, sem_sa, sem_sb):
    c = lax.axis_index("c")
    s = lax.axis_index("s")
    wid = s * NC + c
    zero16 = jnp.zeros((16,), jnp.float32)
    lane = lax.iota(jnp.int32, 16)

    def zero_er(i, _):
        for q in range(D // 16):
            erows[0][i, pl.ds(q * 16, 16)] = zero16
        return 0
    lax.fori_loop(0, CH, zero_er, 0)

    def zero_minis(i, _):
        for q in range(D // 16):
            miniA[i, pl.ds(q * 16, 16)] = zero16
        return 0
    lax.fori_loop(0, NA, zero_minis, 0)

    def zero_minib(i, _):
        for q in range(D // 16):
            miniB[i, pl.ds(q * 16, 16)] = zero16
        return 0
    lax.fori_loop(0, NB, zero_minib, 0)

    for j in range(ROWS_PER_TILE // CH):
        pltpu.sync_copy(erows[0],
                        nh_sh.at[pl.ds(s * ROWS_PER_TILE + j * CH, CH)])

    @pl.when(s < ATILES)
    def _():
        pltpu.sync_copy(erows[0].at[pl.ds(0, ARPT)],
                        att_sh.at[pl.ds(s * ARPT, ARPT)])
    plsc.subcore_barrier()

    def idx_issue(j, b):
        row = wid * NCHUNK + j
        pltpu.async_copy(sde_hbm.at[row], sde_v[b], sem_idx[b])

    def idx_finish(j, b):
        row = wid * NCHUNK + j
        pltpu.make_async_copy(sde_hbm.at[row], sde_v[b],
                              sem_idx[b]).wait()
        for k in range(CH // 16):
            d16 = sde_v[b][pl.ds(CH + k * 16, 16)]
            gidx_v[b][pl.ds(k * 16, 16)] = (
                sde_v[b][pl.ds(2 * CH + k * 16, 16)] * N + d16)
            arow_v[b][pl.ds(k * 16, 16)] = lax.shift_right_logical(d16, 7)

    def egather_start(b):
        pltpu.async_copy(emb_hbm.at[sde_v[b].at[pl.ds(0, CH)]], erows[b],
                         sem_ge[b])

    def ugather_start(b):
        pltpu.async_copy(u_hbm.at[gidx_v[b]], urows, sem_gu)

    def gather_wait(b):
        pltpu.make_async_copy(emb_hbm.at[sde_v[b].at[pl.ds(0, CH)]],
                              erows[b], sem_ge[b]).wait()
        pltpu.make_async_copy(u_hbm.at[gidx_v[b]], urows, sem_gu).wait()

    def dot_edge(b, e):
        eq = tuple(erows[b][e, pl.ds(q * 16, 16)] for q in range(D // 16))
        acc = jnp.zeros((16,), jnp.float32)
        for q in range(D // 16):
            acc = acc + eq[q] * urows[e, pl.ds(q * 16, 16)]
        ae = jnp.exp(jnp.broadcast_to(jnp.sum(acc), (16,)))
        return eq, ae

    def scale_edge(b, e, eq, ae):
        for q in range(D // 16):
            erows[b][e, pl.ds(q * 16, 16)] = eq[q] * ae

    def groups_range(b, lo, hi, mini, off, laneS, arowX):
        def gbody(k, _):
            base_e = k * 16
            eq0, ae0 = dot_edge(b, base_e)

            def ebody(jj, carry):
                ae16, ae_p = carry[0], carry[1]
                eq_p = carry[2:]
                e = base_e + jj
                eqn, aen = dot_edge(b, e)
                scale_edge(b, e - 1, eq_p, ae_p)
                ae16 = jnp.where(lane == (jj - 1), ae_p, ae16)
                return (ae16, aen) + eqn

            carry = lax.fori_loop(1, 16, ebody, (zero16, ae0) + eq0,
                                  unroll=2)
            ae16, ae_l = carry[0], carry[1]
            eq_l = carry[2:]
            scale_edge(b, base_e + 15, eq_l, ae_l)
            ae16 = jnp.where(lane == 15, ae_l, ae16)
            rows_local = lane + (base_e - off)
            d16 = sde_v[b][pl.ds(CH + base_e, 16)]
            dstS_v[b][pl.ds(base_e, 16)] = d16
            lane16 = jnp.bitwise_and(d16, 127)
            plsc.store_scatter(mini, [rows_local, lane16], ae16)
            laneS[pl.ds(base_e - off, 16)] = lane16
            arowX[pl.ds(base_e - off, 16)] = arow_v[b][pl.ds(base_e, 16)]
            return 0
        lax.fori_loop(lo, hi, gbody, 0)

    def clear_mini(mini, nrows, laneS):
        for k in range(nrows // 16):
            rows_local = lane + k * 16
            lane16 = laneS[pl.ds(k * 16, 16)]
            plsc.store_scatter(mini, [rows_local, lane16], zero16)

    def substep(j, b, first, issue_next, issue_u):
        gather_wait(b)
        if not first:
            pltpu.make_async_copy(miniA, att_sh.at[arowA_v], sem_sa).wait()
            clear_mini(miniA, NA, laneA_v)
        groups_range(b, 0, 3, miniA, 0, laneA_v, arowA_v)
        pltpu.async_copy(miniA, att_sh.at[arowA_v], sem_sa, add=True)
        if not first:
            pltpu.make_async_copy(miniB, att_sh.at[arowB_v], sem_sb).wait()
            clear_mini(miniB, NB, laneB_v)
        groups_range(b, 3, CH // 16, miniB, NA, laneB_v, arowB_v)
        pltpu.async_copy(miniB, att_sh.at[arowB_v], sem_sb, add=True)
        if issue_u:
            ugather_start(1 - b)
        pltpu.async_copy(erows[b], nh_sh.at[dstS_v[b]], sem_snh[b], add=True)
        if issue_next:
            idx_issue(j + 2, b)
        pltpu.make_async_copy(erows[b], nh_sh.at[dstS_v[b]],
                              sem_snh[b]).wait()
        if issue_next:
            idx_finish(j + 2, b)
            egather_start(b)

    idx_issue(0, 0)
    idx_finish(0, 0)
    egather_start(0)
    ugather_start(0)
    idx_issue(1, 1)
    idx_finish(1, 1)
    egather_start(1)
    substep(0, 0, True, True, True)
    substep(1, 1, False, True, True)

    def pair_body(i, _):
        substep(2 * i, 0, False, True, True)
        substep(2 * i + 1, 1, False, True, True)
        return 0
    lax.fori_loop(1, (NCHUNK - 3) // 2, pair_body, 0)

    substep(NCHUNK - 3, 0, False, True, True)
    substep(NCHUNK - 2, 1, False, False, True)
    substep(NCHUNK - 1, 0, False, False, False)
    pltpu.make_async_copy(miniA, att_sh.at[arowA_v], sem_sa).wait()
    pltpu.make_async_copy(miniB, att_sh.at[arowB_v], sem_sb).wait()

    plsc.subcore_barrier()
    pltpu.sync_copy(nh_sh.at[pl.ds(s * ROWS_PER_TILE, ROWS_PER_TILE)],
                    nh_out.at[c, pl.ds(s * ROWS_PER_TILE, ROWS_PER_TILE)])

    @pl.when(s < ATILES)
    def _():
        pltpu.sync_copy(att_sh.at[pl.ds(s * ARPT, ARPT)],
                        att_out.at[c, pl.ds(s * ARPT, ARPT)])


@functools.partial(
    pl.kernel,
    out_type=[jax.ShapeDtypeStruct((NC, NP, D), jnp.float32),
              jax.ShapeDtypeStruct((NC, AR, D), jnp.float32)],
    mesh=plsc.VectorSubcoreMesh(core_axis_name="c", subcore_axis_name="s"),
    compiler_params=pltpu.CompilerParams(needs_layout_passes=False),
    scratch_types=[
        [pltpu.VMEM((3 * CH,), jnp.int32)] * 2,
        [pltpu.VMEM((CH,), jnp.int32)] * 2,
        [pltpu.VMEM((CH,), jnp.int32)] * 2,
        [pltpu.VMEM((CH,), jnp.int32)] * 2,
        pltpu.VMEM((NA,), jnp.int32),
        pltpu.VMEM((NB,), jnp.int32),
        pltpu.VMEM((NA,), jnp.int32),
        pltpu.VMEM((NB,), jnp.int32),
        [pltpu.VMEM((CH, D), jnp.float32)] * 2,
        pltpu.VMEM((CH, D), jnp.float32),
        pltpu.VMEM((NA, D), jnp.float32),
        pltpu.VMEM((NB, D), jnp.float32),
        pltpu.VMEM_SHARED((NP, D), jnp.float32),
        pltpu.VMEM_SHARED((AR, D), jnp.float32),
        [pltpu.SemaphoreType.DMA] * 2,
        pltpu.SemaphoreType.DMA,
        [pltpu.SemaphoreType.DMA] * 2,
        [pltpu.SemaphoreType.DMA] * 2,
        pltpu.SemaphoreType.DMA,
        pltpu.SemaphoreType.DMA,
    ],
)
def _sc_edge_pass(emb_hbm, u_hbm, sde_hbm, nh_out, att_out,
                  sde_v, gidx_v, arow_v, dstS_v,
                  laneA_v, laneB_v, arowA_v, arowB_v,
                  erows, urows, miniA, miniB, nh_sh, att_sh,
                  sem_ge, sem_gu, sem_snh, sem_idx, sem_sa, sem_sb):
    _sc_body(emb_hbm, u_hbm, sde_hbm, nh_out, att_out,
             sde_v, gidx_v, arow_v, dstS_v,
             laneA_v, laneB_v, arowA_v, arowB_v,
             erows, urows, miniA, miniB, nh_sh, att_sh,
             sem_ge, sem_gu, sem_snh, sem_idx, sem_sa, sem_sb)


def _fin_body(emb_ref, nh_ref, att_ref, w1t_ref, b1_ref, w2t_ref, b2_ref,
              o_ref):
    att_sum = att_ref[0] + att_ref[1]
    nh = (nh_ref[0] + nh_ref[1]) / (att_sum + 1e-10)
    e = emb_ref[...]
    h1 = jnp.dot(e + nh, w1t_ref[...],
                 preferred_element_type=jnp.float32) + b1_ref[...]
    h2 = jnp.dot(e * nh, w2t_ref[...],
                 preferred_element_type=jnp.float32) + b2_ref[...]
    h1 = jnp.where(h1 > 0, h1, 0.01 * h1)
    h2 = jnp.where(h2 > 0, h2, 0.01 * h2)
    o = h1 + h2
    nrm = jnp.sqrt(jnp.sum(o * o, axis=1, keepdims=True))
    o_ref[...] = o / jnp.maximum(nrm, 1e-12)


def _finalize(emb_p, nh, att, W1t, b1, W2t, b2):
    OUT = W1t.shape[1]
    return pl.pallas_call(
        _fin_body,
        grid=(NP // FB,),
        in_specs=[
            pl.BlockSpec((FB, D), lambda i: (i, 0)),
            pl.BlockSpec((NC, FB, D), lambda i: (0, i, 0)),
            pl.BlockSpec((NC, FB, 1), lambda i: (0, i, 0)),
            pl.BlockSpec((D, OUT), lambda i: (0, 0)),
            pl.BlockSpec((1, OUT), lambda i: (0, 0)),
            pl.BlockSpec((D, OUT), lambda i: (0, 0)),
            pl.BlockSpec((1, OUT), lambda i: (0, 0)),
        ],
        out_specs=pl.BlockSpec((FB, OUT), lambda i: (i, 0)),
        out_shape=jax.ShapeDtypeStruct((NP, OUT), jnp.float32),
    )(emb_p, nh, att, W1t, b1, W2t, b2)


def kernel(node_ids, edge_index, edge_type, emb_table, rel_embed, W_R,
           W1, b1, W2, b2):
    emb = jnp.take(emb_table, node_ids, axis=0)
    u = _precompute_u(emb, rel_embed, W_R)
    sde = (jnp.concatenate([edge_index, edge_type[None]], axis=0)
           .reshape(3, NWORK * NCHUNK, CH).transpose(1, 0, 2)
           .reshape(NWORK * NCHUNK, 3 * CH))
    nh, att = _sc_edge_pass(emb, u, sde)
    att_r = att.reshape(NC, NP, 1)
    emb_p = jnp.concatenate(
        [emb, jnp.zeros((NP - N, D), jnp.float32)], axis=0)
    norm_embed = _finalize(emb_p, nh, att_r, W1.T, b1.reshape(1, -1),
                           W2.T, b2.reshape(1, -1))[:N]
    return jnp.concatenate([emb, norm_embed], axis=1)


# speedup vs baseline: 1.0588x; 1.0033x over previous
import functools

import jax
import jax.numpy as jnp
import numpy as np
from jax import lax
from jax.experimental import pallas as pl
from jax.experimental.pallas import tpu as pltpu
from jax.experimental.pallas import tpu_sc as plsc

N = 10000
E = 320000
R = 8
D = 128
NP = 10240
AR = NP // D
NC = 2
NS = 16
NWORK = NC * NS
EPW = E // NWORK
CH = 80
NCHUNK = EPW // CH
ROWS_PER_TILE = NP // NS
ARPT = 8
ATILES = AR // ARPT
FB = 1024
NA = 48
NB = CH - NA

def _u_body(emb_ref, rel_ref, wr_ref, u_ref):
    w = wr_ref[0]
    p = jnp.dot(emb_ref[...], w, preferred_element_type=jnp.float32)
    t = jnp.tanh(p + rel_ref[0])
    u_ref[...] = lax.dot_general(
        t, w, (((1,), (1,)), ((), ())),
        preferred_element_type=jnp.float32)


def _precompute_u(emb, rel_embed, W_R):
    B = 1000
    grid = (R, N // B)
    return pl.pallas_call(
        _u_body,
        grid=grid,
        in_specs=[
            pl.BlockSpec((B, D), lambda r, i: (i, 0)),
            pl.BlockSpec((1, 1, D), lambda r, i: (r, 0, 0)),
            pl.BlockSpec((1, D, D), lambda r, i: (r, 0, 0)),
        ],
        out_specs=pl.BlockSpec((B, D), lambda r, i: (r * (N // B) + i, 0)),
        out_shape=jax.ShapeDtypeStruct((R * N, D), jnp.float32),
    )(emb, rel_embed.reshape(R, 1, D), W_R)


def _sc_body(emb_hbm, u_hbm, sde_hbm, nh_out, att_out,
             sde_v, gidx_v, arow_v, dstSA_v, dstSB_v,
             laneA_v, laneB_v, arowA_v, arowB_v,
             erows, urows, miniA, miniB, nh_sh, att_sh,
             sem_ge, sem_gu, sem_snhA, sem_snhB, sem_idx, sem_sa, sem_sb):
    c = lax.axis_index("c")
    s = lax.axis_index("s")
    wid = s * NC + c
    zero16 = jnp.zeros((16,), jnp.float32)
    lane = lax.iota(jnp.int32, 16)

    def zero_er(i, _):
        for q in range(D // 16):
            erows[0][i, pl.ds(q * 16, 16)] = zero16
        return 0
    lax.fori_loop(0, CH, zero_er, 0)

    def zero_minis(i, _):
        for q in range(D // 16):
            miniA[i, pl.ds(q * 16, 16)] = zero16
        return 0
    lax.fori_loop(0, NA, zero_minis, 0)

    def zero_minib(i, _):
        for q in range(D // 16):
            miniB[i, pl.ds(q * 16, 16)] = zero16
        return 0
    lax.fori_loop(0, NB, zero_minib, 0)

    for j in range(ROWS_PER_TILE // CH):
        pltpu.sync_copy(erows[0],
                        nh_sh.at[pl.ds(s * ROWS_PER_TILE + j * CH, CH)])

    @pl.when(s < ATILES)
    def _():
        pltpu.sync_copy(erows[0].at[pl.ds(0, ARPT)],
                        att_sh.at[pl.ds(s * ARPT, ARPT)])
    plsc.subcore_barrier()

    def idx_issue(j, b):
        row = wid * NCHUNK + j
        pltpu.async_copy(sde_hbm.at[row], sde_v[b], sem_idx[b])

    def idx_finish(j, b):
        row = wid * NCHUNK + j
        pltpu.make_async_copy(sde_hbm.at[row], sde_v[b],
                              sem_idx[b]).wait()
        for k in range(CH // 16):
            d16 = sde_v[b][pl.ds(CH + k * 16, 16)]
            gidx_v[b][pl.ds(k * 16, 16)] = (
                sde_v[b][pl.ds(2 * CH + k * 16, 16)] * N + d16)
            arow_v[b][pl.ds(k * 16, 16)] = lax.shift_right_logical(d16, 7)

    def egather_start(b):
        pltpu.async_copy(emb_hbm.at[sde_v[b].at[pl.ds(0, CH)]], erows[b],
                         sem_ge[b])

    def ugather_start(b):
        pltpu.async_copy(u_hbm.at[gidx_v[b]], urows, sem_gu)

    def gather_wait(b):
        pltpu.make_async_copy(emb_hbm.at[sde_v[b].at[pl.ds(0, CH)]],
                              erows[b], sem_ge[b]).wait()
        pltpu.make_async_copy(u_hbm.at[gidx_v[b]], urows, sem_gu).wait()

    def dot_edge(b, e):
        eq = tuple(erows[b][e, pl.ds(q * 16, 16)] for q in range(D // 16))
        acc = jnp.zeros((16,), jnp.float32)
        for q in range(D // 16):
            acc = acc + eq[q] * urows[e, pl.ds(q * 16, 16)]
        ae = jnp.exp(jnp.broadcast_to(jnp.sum(acc), (16,)))
        return eq, ae

    def scale_edge(b, e, eq, ae):
        for q in range(D // 16):
            erows[b][e, pl.ds(q * 16, 16)] = eq[q] * ae

    def groups_range(b, lo, hi, mini, off, laneS, arowX, dstS):
        def gbody(k, _):
            base_e = k * 16
            eq0, ae0 = dot_edge(b, base_e)

            def ebody(jj, carry):
                ae16, ae_p = carry[0], carry[1]
                eq_p = carry[2:]
                e = base_e + jj
                eqn, aen = dot_edge(b, e)
                scale_edge(b, e - 1, eq_p, ae_p)
                ae16 = jnp.where(lane == (jj - 1), ae_p, ae16)
                return (ae16, aen) + eqn

            carry = lax.fori_loop(1, 16, ebody, (zero16, ae0) + eq0,
                                  unroll=2)
            ae16, ae_l = carry[0], carry[1]
            eq_l = carry[2:]
            scale_edge(b, base_e + 15, eq_l, ae_l)
            ae16 = jnp.where(lane == 15, ae_l, ae16)
            rows_local = lane + (base_e - off)
            d16 = sde_v[b][pl.ds(CH + base_e, 16)]
            dstS[pl.ds(base_e - off, 16)] = d16
            lane16 = jnp.bitwise_and(d16, 127)
            plsc.store_scatter(mini, [rows_local, lane16], ae16)
            laneS[pl.ds(base_e - off, 16)] = lane16
            arowX[pl.ds(base_e - off, 16)] = arow_v[b][pl.ds(base_e, 16)]
            return 0
        lax.fori_loop(lo, hi, gbody, 0)

    def clear_mini(mini, nrows, laneS):
        for k in range(nrows // 16):
            rows_local = lane + k * 16
            lane16 = laneS[pl.ds(k * 16, 16)]
            plsc.store_scatter(mini, [rows_local, lane16], zero16)

    def substep(j, b, first, issue_next, issue_u):
        gather_wait(b)
        if not first:
            pltpu.make_async_copy(miniA, att_sh.at[arowA_v], sem_sa).wait()
            clear_mini(miniA, NA, laneA_v)
        groups_range(b, 0, 3, miniA, 0, laneA_v, arowA_v, dstSA_v[b])
        pltpu.async_copy(erows[b].at[pl.ds(0, NA)], nh_sh.at[dstSA_v[b]],
                         sem_snhA[b], add=True)
        pltpu.async_copy(miniA, att_sh.at[arowA_v], sem_sa, add=True)
        if not first:
            pltpu.make_async_copy(miniB, att_sh.at[arowB_v], sem_sb).wait()
            clear_mini(miniB, NB, laneB_v)
        groups_range(b, 3, CH // 16, miniB, NA, laneB_v, arowB_v,
                     dstSB_v[b])
        pltpu.async_copy(erows[b].at[pl.ds(NA, NB)], nh_sh.at[dstSB_v[b]],
                         sem_snhB[b], add=True)
        pltpu.async_copy(miniB, att_sh.at[arowB_v], sem_sb, add=True)
        if issue_u:
            ugather_start(1 - b)
        if issue_next:
            idx_issue(j + 2, b)
            idx_finish(j + 2, b)
        pltpu.make_async_copy(erows[b].at[pl.ds(0, NA)],
                              nh_sh.at[dstSA_v[b]], sem_snhA[b]).wait()
        pltpu.make_async_copy(erows[b].at[pl.ds(NA, NB)],
                              nh_sh.at[dstSB_v[b]], sem_snhB[b]).wait()
        if issue_next:
            egather_start(b)

    idx_issue(0, 0)
    idx_finish(0, 0)
    egather_start(0)
    ugather_start(0)
    idx_issue(1, 1)
    idx_finish(1, 1)
    egather_start(1)
    substep(0, 0, True, True, True)
    substep(1, 1, False, True, True)

    def pair_body(i, _):
        substep(2 * i, 0, False, True, True)
        substep(2 * i + 1, 1, False, True, True)
        return 0
    lax.fori_loop(1, (NCHUNK - 3) // 2, pair_body, 0)

    substep(NCHUNK - 3, 0, False, True, True)
    substep(NCHUNK - 2, 1, False, False, True)
    substep(NCHUNK - 1, 0, False, False, False)
    pltpu.make_async_copy(miniA, att_sh.at[arowA_v], sem_sa).wait()
    pltpu.make_async_copy(miniB, att_sh.at[arowB_v], sem_sb).wait()

    plsc.subcore_barrier()
    pltpu.sync_copy(nh_sh.at[pl.ds(s * ROWS_PER_TILE, ROWS_PER_TILE)],
                    nh_out.at[c, pl.ds(s * ROWS_PER_TILE, ROWS_PER_TILE)])

    @pl.when(s < ATILES)
    def _():
        pltpu.sync_copy(att_sh.at[pl.ds(s * ARPT, ARPT)],
                        att_out.at[c, pl.ds(s * ARPT, ARPT)])


@functools.partial(
    pl.kernel,
    out_type=[jax.ShapeDtypeStruct((NC, NP, D), jnp.float32),
              jax.ShapeDtypeStruct((NC, AR, D), jnp.float32)],
    mesh=plsc.VectorSubcoreMesh(core_axis_name="c", subcore_axis_name="s"),
    compiler_params=pltpu.CompilerParams(needs_layout_passes=False),
    scratch_types=[
        [pltpu.VMEM((3 * CH,), jnp.int32)] * 2,
        [pltpu.VMEM((CH,), jnp.int32)] * 2,
        [pltpu.VMEM((CH,), jnp.int32)] * 2,
        [pltpu.VMEM((NA,), jnp.int32)] * 2,
        [pltpu.VMEM((NB,), jnp.int32)] * 2,
        pltpu.VMEM((NA,), jnp.int32),
        pltpu.VMEM((NB,), jnp.int32),
        pltpu.VMEM((NA,), jnp.int32),
        pltpu.VMEM((NB,), jnp.int32),
        [pltpu.VMEM((CH, D), jnp.float32)] * 2,
        pltpu.VMEM((CH, D), jnp.float32),
        pltpu.VMEM((NA, D), jnp.float32),
        pltpu.VMEM((NB, D), jnp.float32),
        pltpu.VMEM_SHARED((NP, D), jnp.float32),
        pltpu.VMEM_SHARED((AR, D), jnp.float32),
        [pltpu.SemaphoreType.DMA] * 2,
        pltpu.SemaphoreType.DMA,
        [pltpu.SemaphoreType.DMA] * 2,
        [pltpu.SemaphoreType.DMA] * 2,
        [pltpu.SemaphoreType.DMA] * 2,
        pltpu.SemaphoreType.DMA,
        pltpu.SemaphoreType.DMA,
    ],
)
def _sc_edge_pass(emb_hbm, u_hbm, sde_hbm, nh_out, att_out,
                  sde_v, gidx_v, arow_v, dstSA_v, dstSB_v,
                  laneA_v, laneB_v, arowA_v, arowB_v,
                  erows, urows, miniA, miniB, nh_sh, att_sh,
                  sem_ge, sem_gu, sem_snhA, sem_snhB, sem_idx, sem_sa,
                  sem_sb):
    _sc_body(emb_hbm, u_hbm, sde_hbm, nh_out, att_out,
             sde_v, gidx_v, arow_v, dstSA_v, dstSB_v,
             laneA_v, laneB_v, arowA_v, arowB_v,
             erows, urows, miniA, miniB, nh_sh, att_sh,
             sem_ge, sem_gu, sem_snhA, sem_snhB, sem_idx, sem_sa, sem_sb)


def _fin_body(emb_ref, nh_ref, att_ref, w1t_ref, b1_ref, w2t_ref, b2_ref,
              o_ref):
    att_sum = att_ref[0] + att_ref[1]
    nh = (nh_ref[0] + nh_ref[1]) / (att_sum + 1e-10)
    e = emb_ref[...]
    h1 = jnp.dot(e + nh, w1t_ref[...],
                 preferred_element_type=jnp.float32) + b1_ref[...]
    h2 = jnp.dot(e * nh, w2t_ref[...],
                 preferred_element_type=jnp.float32) + b2_ref[...]
    h1 = jnp.where(h1 > 0, h1, 0.01 * h1)
    h2 = jnp.where(h2 > 0, h2, 0.01 * h2)
    o = h1 + h2
    nrm = jnp.sqrt(jnp.sum(o * o, axis=1, keepdims=True))
    o_ref[...] = o / jnp.maximum(nrm, 1e-12)


def _finalize(emb_p, nh, att, W1t, b1, W2t, b2):
    OUT = W1t.shape[1]
    return pl.pallas_call(
        _fin_body,
        grid=(NP // FB,),
        in_specs=[
            pl.BlockSpec((FB, D), lambda i: (i, 0)),
            pl.BlockSpec((NC, FB, D), lambda i: (0, i, 0)),
            pl.BlockSpec((NC, FB, 1), lambda i: (0, i, 0)),
            pl.BlockSpec((D, OUT), lambda i: (0, 0)),
            pl.BlockSpec((1, OUT), lambda i: (0, 0)),
            pl.BlockSpec((D, OUT), lambda i: (0, 0)),
            pl.BlockSpec((1, OUT), lambda i: (0, 0)),
        ],
        out_specs=pl.BlockSpec((FB, OUT), lambda i: (i, 0)),
        out_shape=jax.ShapeDtypeStruct((NP, OUT), jnp.float32),
    )(emb_p, nh, att, W1t, b1, W2t, b2)


def kernel(node_ids, edge_index, edge_type, emb_table, rel_embed, W_R,
           W1, b1, W2, b2):
    emb = jnp.take(emb_table, node_ids, axis=0)
    u = _precompute_u(emb, rel_embed, W_R)
    sde = (jnp.concatenate([edge_index, edge_type[None]], axis=0)
           .reshape(3, NWORK * NCHUNK, CH).transpose(1, 0, 2)
           .reshape(NWORK * NCHUNK, 3 * CH))
    nh, att = _sc_edge_pass(emb, u, sde)
    att_r = att.reshape(NC, NP, 1)
    emb_p = jnp.concatenate(
        [emb, jnp.zeros((NP - N, D), jnp.float32)], axis=0)
    norm_embed = _finalize(emb_p, nh, att_r, W1.T, b1.reshape(1, -1),
                           W2.T, b2.reshape(1, -1))[:N]
    return jnp.concatenate([emb, norm_embed], axis=1)


# speedup vs baseline: 1.1238x; 1.0614x over previous
import functools

import jax
import jax.numpy as jnp
import numpy as np
from jax import lax
from jax.experimental import pallas as pl
from jax.experimental.pallas import tpu as pltpu
from jax.experimental.pallas import tpu_sc as plsc

N = 10000
E = 320000
R = 8
D = 128
NP = 10240
AR = NP // D
NC = 2
NS = 16
NWORK = NC * NS
EPW = E // NWORK
CH = 80
NCHUNK = EPW // CH
ROWS_PER_TILE = NP // NS
ARPT = 8
ATILES = AR // ARPT
FB = 1024
NA = 48
NB = CH - NA

def _u_body(emb_ref, rel_ref, wr_ref, u_ref):
    w = wr_ref[0]
    p = jnp.dot(emb_ref[...], w, preferred_element_type=jnp.float32)
    t = jnp.tanh(p + rel_ref[0])
    u_ref[...] = lax.dot_general(
        t, w, (((1,), (1,)), ((), ())),
        preferred_element_type=jnp.float32)


def _precompute_u(emb, rel_embed, W_R):
    B = 1000
    grid = (R, N // B)
    return pl.pallas_call(
        _u_body,
        grid=grid,
        in_specs=[
            pl.BlockSpec((B, D), lambda r, i: (i, 0)),
            pl.BlockSpec((1, 1, D), lambda r, i: (r, 0, 0)),
            pl.BlockSpec((1, D, D), lambda r, i: (r, 0, 0)),
        ],
        out_specs=pl.BlockSpec((B, D), lambda r, i: (r * (N // B) + i, 0)),
        out_shape=jax.ShapeDtypeStruct((R * N, D), jnp.float32),
    )(emb, rel_embed.reshape(R, 1, D), W_R)


def _sc_body(emb_hbm, u_hbm, sde_hbm, nh_out, att_out,
             sde_v, gidx_v, arow_v, dstSA_v, dstSB_v,
             laneA_v, laneB_v, arowA_v, arowB_v,
             erows, urows, miniA, miniB, nh_sh, att_sh,
             sem_ge, sem_guA, sem_guB, sem_snhA, sem_snhB, sem_idx, sem_sa, sem_sb):
    c = lax.axis_index("c")
    s = lax.axis_index("s")
    wid = s * NC + c
    zero16 = jnp.zeros((16,), jnp.float32)
    lane = lax.iota(jnp.int32, 16)

    def zero_er(i, _):
        for q in range(D // 16):
            erows[0][i, pl.ds(q * 16, 16)] = zero16
        return 0
    lax.fori_loop(0, CH, zero_er, 0)

    def zero_minis(i, _):
        for q in range(D // 16):
            miniA[i, pl.ds(q * 16, 16)] = zero16
        return 0
    lax.fori_loop(0, NA, zero_minis, 0)

    def zero_minib(i, _):
        for q in range(D // 16):
            miniB[i, pl.ds(q * 16, 16)] = zero16
        return 0
    lax.fori_loop(0, NB, zero_minib, 0)

    for j in range(ROWS_PER_TILE // CH):
        pltpu.sync_copy(erows[0],
                        nh_sh.at[pl.ds(s * ROWS_PER_TILE + j * CH, CH)])

    @pl.when(s < ATILES)
    def _():
        pltpu.sync_copy(erows[0].at[pl.ds(0, ARPT)],
                        att_sh.at[pl.ds(s * ARPT, ARPT)])
    plsc.subcore_barrier()

    def idx_issue(j, b):
        row = wid * NCHUNK + j
        pltpu.async_copy(sde_hbm.at[row], sde_v[b], sem_idx[b])

    def idx_finish(j, b):
        row = wid * NCHUNK + j
        pltpu.make_async_copy(sde_hbm.at[row], sde_v[b],
                              sem_idx[b]).wait()
        for k in range(CH // 16):
            d16 = sde_v[b][pl.ds(CH + k * 16, 16)]
            gidx_v[b][pl.ds(k * 16, 16)] = (
                sde_v[b][pl.ds(2 * CH + k * 16, 16)] * N + d16)
            arow_v[b][pl.ds(k * 16, 16)] = lax.shift_right_logical(d16, 7)

    def egather_start(b):
        pltpu.async_copy(emb_hbm.at[sde_v[b].at[pl.ds(0, CH)]], erows[b],
                         sem_ge[b])

    def ugatherA_start(b):
        pltpu.async_copy(u_hbm.at[gidx_v[b].at[pl.ds(0, NA)]],
                         urows.at[pl.ds(0, NA)], sem_guA)

    def ugatherB_start(b):
        pltpu.async_copy(u_hbm.at[gidx_v[b].at[pl.ds(NA, NB)]],
                         urows.at[pl.ds(NA, NB)], sem_guB)

    def gather_wait(b):
        pltpu.make_async_copy(emb_hbm.at[sde_v[b].at[pl.ds(0, CH)]],
                              erows[b], sem_ge[b]).wait()
        pltpu.make_async_copy(u_hbm.at[gidx_v[b].at[pl.ds(0, NA)]],
                              urows.at[pl.ds(0, NA)], sem_guA).wait()
        pltpu.make_async_copy(u_hbm.at[gidx_v[b].at[pl.ds(NA, NB)]],
                              urows.at[pl.ds(NA, NB)], sem_guB).wait()

    def dot_edge(b, e):
        eq = tuple(erows[b][e, pl.ds(q * 16, 16)] for q in range(D // 16))
        acc = jnp.zeros((16,), jnp.float32)
        for q in range(D // 16):
            acc = acc + eq[q] * urows[e, pl.ds(q * 16, 16)]
        ae = jnp.exp(jnp.broadcast_to(jnp.sum(acc), (16,)))
        return eq, ae

    def scale_edge(b, e, eq, ae):
        for q in range(D // 16):
            erows[b][e, pl.ds(q * 16, 16)] = eq[q] * ae

    def groups_range(b, lo, hi, mini, off, laneS, arowX, dstS):
        def gbody(k, _):
            base_e = k * 16
            eq0, ae0 = dot_edge(b, base_e)

            def ebody(jj, carry):
                ae16, ae_p = carry[0], carry[1]
                eq_p = carry[2:]
                e = base_e + jj
                eqn, aen = dot_edge(b, e)
                scale_edge(b, e - 1, eq_p, ae_p)
                ae16 = jnp.where(lane == (jj - 1), ae_p, ae16)
                return (ae16, aen) + eqn

            carry = lax.fori_loop(1, 16, ebody, (zero16, ae0) + eq0,
                                  unroll=2)
            ae16, ae_l = carry[0], carry[1]
            eq_l = carry[2:]
            scale_edge(b, base_e + 15, eq_l, ae_l)
            ae16 = jnp.where(lane == 15, ae_l, ae16)
            rows_local = lane + (base_e - off)
            d16 = sde_v[b][pl.ds(CH + base_e, 16)]
            dstS[pl.ds(base_e - off, 16)] = d16
            lane16 = jnp.bitwise_and(d16, 127)
            plsc.store_scatter(mini, [rows_local, lane16], ae16)
            laneS[pl.ds(base_e - off, 16)] = lane16
            arowX[pl.ds(base_e - off, 16)] = arow_v[b][pl.ds(base_e, 16)]
            return 0
        lax.fori_loop(lo, hi, gbody, 0)

    def clear_mini(mini, nrows, laneS):
        for k in range(nrows // 16):
            rows_local = lane + k * 16
            lane16 = laneS[pl.ds(k * 16, 16)]
            plsc.store_scatter(mini, [rows_local, lane16], zero16)

    def substep(j, b, first, issue_next, issue_u):
        gather_wait(b)
        if not first:
            pltpu.make_async_copy(miniA, att_sh.at[arowA_v], sem_sa).wait()
            clear_mini(miniA, NA, laneA_v)
        groups_range(b, 0, 3, miniA, 0, laneA_v, arowA_v, dstSA_v[b])
        pltpu.async_copy(erows[b].at[pl.ds(0, NA)], nh_sh.at[dstSA_v[b]],
                         sem_snhA[b], add=True)
        pltpu.async_copy(miniA, att_sh.at[arowA_v], sem_sa, add=True)
        if not first:
            pltpu.make_async_copy(miniB, att_sh.at[arowB_v], sem_sb).wait()
            clear_mini(miniB, NB, laneB_v)
        if issue_u:
            ugatherA_start(1 - b)
        groups_range(b, 3, CH // 16, miniB, NA, laneB_v, arowB_v,
                     dstSB_v[b])
        pltpu.async_copy(erows[b].at[pl.ds(NA, NB)], nh_sh.at[dstSB_v[b]],
                         sem_snhB[b], add=True)
        pltpu.async_copy(miniB, att_sh.at[arowB_v], sem_sb, add=True)
        if issue_u:
            ugatherB_start(1 - b)
        if issue_next:
            idx_issue(j + 2, b)
            idx_finish(j + 2, b)
        pltpu.make_async_copy(erows[b].at[pl.ds(0, NA)],
                              nh_sh.at[dstSA_v[b]], sem_snhA[b]).wait()
        pltpu.make_async_copy(erows[b].at[pl.ds(NA, NB)],
                              nh_sh.at[dstSB_v[b]], sem_snhB[b]).wait()
        if issue_next:
            egather_start(b)

    idx_issue(0, 0)
    idx_finish(0, 0)
    egather_start(0)
    ugatherA_start(0)
    ugatherB_start(0)
    idx_issue(1, 1)
    idx_finish(1, 1)
    egather_start(1)
    substep(0, 0, True, True, True)
    substep(1, 1, False, True, True)

    def pair_body(i, _):
        substep(2 * i, 0, False, True, True)
        substep(2 * i + 1, 1, False, True, True)
        return 0
    lax.fori_loop(1, (NCHUNK - 3) // 2, pair_body, 0)

    substep(NCHUNK - 3, 0, False, True, True)
    substep(NCHUNK - 2, 1, False, False, True)
    substep(NCHUNK - 1, 0, False, False, False)
    pltpu.make_async_copy(miniA, att_sh.at[arowA_v], sem_sa).wait()
    pltpu.make_async_copy(miniB, att_sh.at[arowB_v], sem_sb).wait()

    plsc.subcore_barrier()
    pltpu.sync_copy(nh_sh.at[pl.ds(s * ROWS_PER_TILE, ROWS_PER_TILE)],
                    nh_out.at[c, pl.ds(s * ROWS_PER_TILE, ROWS_PER_TILE)])

    @pl.when(s < ATILES)
    def _():
        pltpu.sync_copy(att_sh.at[pl.ds(s * ARPT, ARPT)],
                        att_out.at[c, pl.ds(s * ARPT, ARPT)])


@functools.partial(
    pl.kernel,
    out_type=[jax.ShapeDtypeStruct((NC, NP, D), jnp.float32),
              jax.ShapeDtypeStruct((NC, AR, D), jnp.float32)],
    mesh=plsc.VectorSubcoreMesh(core_axis_name="c", subcore_axis_name="s"),
    compiler_params=pltpu.CompilerParams(needs_layout_passes=False),
    scratch_types=[
        [pltpu.VMEM((3 * CH,), jnp.int32)] * 2,
        [pltpu.VMEM((CH,), jnp.int32)] * 2,
        [pltpu.VMEM((CH,), jnp.int32)] * 2,
        [pltpu.VMEM((NA,), jnp.int32)] * 2,
        [pltpu.VMEM((NB,), jnp.int32)] * 2,
        pltpu.VMEM((NA,), jnp.int32),
        pltpu.VMEM((NB,), jnp.int32),
        pltpu.VMEM((NA,), jnp.int32),
        pltpu.VMEM((NB,), jnp.int32),
        [pltpu.VMEM((CH, D), jnp.float32)] * 2,
        pltpu.VMEM((CH, D), jnp.float32),
        pltpu.VMEM((NA, D), jnp.float32),
        pltpu.VMEM((NB, D), jnp.float32),
        pltpu.VMEM_SHARED((NP, D), jnp.float32),
        pltpu.VMEM_SHARED((AR, D), jnp.float32),
        [pltpu.SemaphoreType.DMA] * 2,
        pltpu.SemaphoreType.DMA,
        pltpu.SemaphoreType.DMA,
        [pltpu.SemaphoreType.DMA] * 2,
        [pltpu.SemaphoreType.DMA] * 2,
        [pltpu.SemaphoreType.DMA] * 2,
        pltpu.SemaphoreType.DMA,
        pltpu.SemaphoreType.DMA,
    ],
)
def _sc_edge_pass(emb_hbm, u_hbm, sde_hbm, nh_out, att_out,
                  sde_v, gidx_v, arow_v, dstSA_v, dstSB_v,
                  laneA_v, laneB_v, arowA_v, arowB_v,
                  erows, urows, miniA, miniB, nh_sh, att_sh,
                  sem_ge, sem_guA, sem_guB, sem_snhA, sem_snhB, sem_idx, sem_sa,
                  sem_sb):
    _sc_body(emb_hbm, u_hbm, sde_hbm, nh_out, att_out,
             sde_v, gidx_v, arow_v, dstSA_v, dstSB_v,
             laneA_v, laneB_v, arowA_v, arowB_v,
             erows, urows, miniA, miniB, nh_sh, att_sh,
             sem_ge, sem_guA, sem_guB, sem_snhA, sem_snhB, sem_idx, sem_sa, sem_sb)


def _fin_body(emb_ref, nh_ref, att_ref, w1t_ref, b1_ref, w2t_ref, b2_ref,
              o_ref):
    att_sum = att_ref[0] + att_ref[1]
    nh = (nh_ref[0] + nh_ref[1]) / (att_sum + 1e-10)
    e = emb_ref[...]
    h1 = jnp.dot(e + nh, w1t_ref[...],
                 preferred_element_type=jnp.float32) + b1_ref[...]
    h2 = jnp.dot(e * nh, w2t_ref[...],
                 preferred_element_type=jnp.float32) + b2_ref[...]
    h1 = jnp.where(h1 > 0, h1, 0.01 * h1)
    h2 = jnp.where(h2 > 0, h2, 0.01 * h2)
    o = h1 + h2
    nrm = jnp.sqrt(jnp.sum(o * o, axis=1, keepdims=True))
    o_ref[...] = o / jnp.maximum(nrm, 1e-12)


def _finalize(emb_p, nh, att, W1t, b1, W2t, b2):
    OUT = W1t.shape[1]
    return pl.pallas_call(
        _fin_body,
        grid=(NP // FB,),
        in_specs=[
            pl.BlockSpec((FB, D), lambda i: (i, 0)),
            pl.BlockSpec((NC, FB, D), lambda i: (0, i, 0)),
            pl.BlockSpec((NC, FB, 1), lambda i: (0, i, 0)),
            pl.BlockSpec((D, OUT), lambda i: (0, 0)),
            pl.BlockSpec((1, OUT), lambda i: (0, 0)),
            pl.BlockSpec((D, OUT), lambda i: (0, 0)),
            pl.BlockSpec((1, OUT), lambda i: (0, 0)),
        ],
        out_specs=pl.BlockSpec((FB, OUT), lambda i: (i, 0)),
        out_shape=jax.ShapeDtypeStruct((NP, OUT), jnp.float32),
    )(emb_p, nh, att, W1t, b1, W2t, b2)


def kernel(node_ids, edge_index, edge_type, emb_table, rel_embed, W_R,
           W1, b1, W2, b2):
    emb = jnp.take(emb_table, node_ids, axis=0)
    u = _precompute_u(emb, rel_embed, W_R)
    sde = (jnp.concatenate([edge_index, edge_type[None]], axis=0)
           .reshape(3, NWORK * NCHUNK, CH).transpose(1, 0, 2)
           .reshape(NWORK * NCHUNK, 3 * CH))
    nh, att = _sc_edge_pass(emb, u, sde)
    att_r = att.reshape(NC, NP, 1)
    emb_p = jnp.concatenate(
        [emb, jnp.zeros((NP - N, D), jnp.float32)], axis=0)
    norm_embed = _finalize(emb_p, nh, att_r, W1.T, b1.reshape(1, -1),
                           W2.T, b2.reshape(1, -1))[:N]
    return jnp.concatenate([emb, norm_embed], axis=1)


# speedup vs baseline: 1.2112x; 1.0778x over previous
import functools

import jax
import jax.numpy as jnp
import numpy as np
from jax import lax
from jax.experimental import pallas as pl
from jax.experimental.pallas import tpu as pltpu
from jax.experimental.pallas import tpu_sc as plsc

N = 10000
E = 320000
R = 8
D = 128
NP = 10240
AR = NP // D
NC = 2
NS = 16
NWORK = NC * NS
EPW = E // NWORK
CH = 80
NCHUNK = EPW // CH
ROWS_PER_TILE = NP // NS
ARPT = 8
ATILES = AR // ARPT
FB = 1000
NA = 48
NB = CH - NA

def _u_body(emb_ref, rel_ref, wr_ref, u_ref):
    w = wr_ref[0]
    p = jnp.dot(emb_ref[...], w, preferred_element_type=jnp.float32)
    t = jnp.tanh(p + rel_ref[0])
    u_ref[...] = lax.dot_general(
        t, w, (((1,), (1,)), ((), ())),
        preferred_element_type=jnp.float32)


def _precompute_u(emb, rel_embed, W_R):
    B = 1000
    grid = (R, N // B)
    return pl.pallas_call(
        _u_body,
        grid=grid,
        in_specs=[
            pl.BlockSpec((B, D), lambda r, i: (i, 0)),
            pl.BlockSpec((1, 1, D), lambda r, i: (r, 0, 0)),
            pl.BlockSpec((1, D, D), lambda r, i: (r, 0, 0)),
        ],
        out_specs=pl.BlockSpec((B, D), lambda r, i: (r * (N // B) + i, 0)),
        out_shape=jax.ShapeDtypeStruct((R * N, D), jnp.float32),
    )(emb, rel_embed.reshape(R, 1, D), W_R)


def _sc_body(emb_hbm, u_hbm, sde_hbm, nh_out, att_out,
             sde_v, gidx_v, arow_v, dstSA_v, dstSB_v,
             laneA_v, laneB_v, arowA_v, arowB_v,
             erows, urows, miniA, miniB, nh_sh, att_sh,
             sem_ge, sem_guA, sem_guB, sem_snhA, sem_snhB, sem_idx, sem_sa, sem_sb):
    c = lax.axis_index("c")
    s = lax.axis_index("s")
    wid = s * NC + c
    zero16 = jnp.zeros((16,), jnp.float32)
    lane = lax.iota(jnp.int32, 16)

    def zero_er(i, _):
        for q in range(D // 16):
            erows[0][i, pl.ds(q * 16, 16)] = zero16
        return 0
    lax.fori_loop(0, CH, zero_er, 0)

    def zero_minis(i, _):
        for q in range(D // 16):
            miniA[i, pl.ds(q * 16, 16)] = zero16
        return 0
    lax.fori_loop(0, NA, zero_minis, 0)

    def zero_minib(i, _):
        for q in range(D // 16):
            miniB[i, pl.ds(q * 16, 16)] = zero16
        return 0
    lax.fori_loop(0, NB, zero_minib, 0)

    for j in range(ROWS_PER_TILE // CH):
        pltpu.sync_copy(erows[0],
                        nh_sh.at[pl.ds(s * ROWS_PER_TILE + j * CH, CH)])

    @pl.when(s < ATILES)
    def _():
        pltpu.sync_copy(erows[0].at[pl.ds(0, ARPT)],
                        att_sh.at[pl.ds(s * ARPT, ARPT)])
    plsc.subcore_barrier()

    def idx_issue(j, b):
        row = wid * NCHUNK + j
        pltpu.async_copy(sde_hbm.at[row], sde_v[b], sem_idx[b])

    def idx_finish(j, b):
        row = wid * NCHUNK + j
        pltpu.make_async_copy(sde_hbm.at[row], sde_v[b],
                              sem_idx[b]).wait()
        for k in range(CH // 16):
            d16 = sde_v[b][pl.ds(CH + k * 16, 16)]
            gidx_v[b][pl.ds(k * 16, 16)] = (
                sde_v[b][pl.ds(2 * CH + k * 16, 16)] * N + d16)
            arow_v[b][pl.ds(k * 16, 16)] = lax.shift_right_logical(d16, 7)

    def egather_start(b):
        pltpu.async_copy(emb_hbm.at[sde_v[b].at[pl.ds(0, CH)]], erows[b],
                         sem_ge[b])

    def ugatherA_start(b):
        pltpu.async_copy(u_hbm.at[gidx_v[b].at[pl.ds(0, NA)]],
                         urows.at[pl.ds(0, NA)], sem_guA)

    def ugatherB_start(b):
        pltpu.async_copy(u_hbm.at[gidx_v[b].at[pl.ds(NA, NB)]],
                         urows.at[pl.ds(NA, NB)], sem_guB)

    def gather_wait(b):
        pltpu.make_async_copy(emb_hbm.at[sde_v[b].at[pl.ds(0, CH)]],
                              erows[b], sem_ge[b]).wait()
        pltpu.make_async_copy(u_hbm.at[gidx_v[b].at[pl.ds(0, NA)]],
                              urows.at[pl.ds(0, NA)], sem_guA).wait()
        pltpu.make_async_copy(u_hbm.at[gidx_v[b].at[pl.ds(NA, NB)]],
                              urows.at[pl.ds(NA, NB)], sem_guB).wait()

    def dot_edge(b, e):
        eq = tuple(erows[b][e, pl.ds(q * 16, 16)] for q in range(D // 16))
        acc = jnp.zeros((16,), jnp.float32)
        for q in range(D // 16):
            acc = acc + eq[q] * urows[e, pl.ds(q * 16, 16)]
        ae = jnp.exp(jnp.broadcast_to(jnp.sum(acc), (16,)))
        return eq, ae

    def scale_edge(b, e, eq, ae):
        for q in range(D // 16):
            erows[b][e, pl.ds(q * 16, 16)] = eq[q] * ae

    def groups_range(b, lo, hi, mini, off, laneS, arowX, dstS):
        def gbody(k, _):
            base_e = k * 16
            eq0, ae0 = dot_edge(b, base_e)

            def ebody(jj, carry):
                ae16, ae_p = carry[0], carry[1]
                eq_p = carry[2:]
                e = base_e + jj
                eqn, aen = dot_edge(b, e)
                scale_edge(b, e - 1, eq_p, ae_p)
                ae16 = jnp.where(lane == (jj - 1), ae_p, ae16)
                return (ae16, aen) + eqn

            carry = lax.fori_loop(1, 16, ebody, (zero16, ae0) + eq0,
                                  unroll=2)
            ae16, ae_l = carry[0], carry[1]
            eq_l = carry[2:]
            scale_edge(b, base_e + 15, eq_l, ae_l)
            ae16 = jnp.where(lane == 15, ae_l, ae16)
            rows_local = lane + (base_e - off)
            d16 = sde_v[b][pl.ds(CH + base_e, 16)]
            dstS[pl.ds(base_e - off, 16)] = d16
            lane16 = jnp.bitwise_and(d16, 127)
            plsc.store_scatter(mini, [rows_local, lane16], ae16)
            laneS[pl.ds(base_e - off, 16)] = lane16
            arowX[pl.ds(base_e - off, 16)] = arow_v[b][pl.ds(base_e, 16)]
            return 0
        lax.fori_loop(lo, hi, gbody, 0)

    def clear_mini(mini, nrows, laneS):
        for k in range(nrows // 16):
            rows_local = lane + k * 16
            lane16 = laneS[pl.ds(k * 16, 16)]
            plsc.store_scatter(mini, [rows_local, lane16], zero16)

    def substep(j, b, first, issue_next, issue_u):
        gather_wait(b)
        if not first:
            pltpu.make_async_copy(miniA, att_sh.at[arowA_v], sem_sa).wait()
            clear_mini(miniA, NA, laneA_v)
        groups_range(b, 0, 3, miniA, 0, laneA_v, arowA_v, dstSA_v[b])
        pltpu.async_copy(erows[b].at[pl.ds(0, NA)], nh_sh.at[dstSA_v[b]],
                         sem_snhA[b], add=True)
        pltpu.async_copy(miniA, att_sh.at[arowA_v], sem_sa, add=True)
        if not first:
            pltpu.make_async_copy(miniB, att_sh.at[arowB_v], sem_sb).wait()
            clear_mini(miniB, NB, laneB_v)
        if issue_u:
            ugatherA_start(1 - b)
        groups_range(b, 3, CH // 16, miniB, NA, laneB_v, arowB_v,
                     dstSB_v[b])
        pltpu.async_copy(erows[b].at[pl.ds(NA, NB)], nh_sh.at[dstSB_v[b]],
                         sem_snhB[b], add=True)
        pltpu.async_copy(miniB, att_sh.at[arowB_v], sem_sb, add=True)
        if issue_u:
            ugatherB_start(1 - b)
        if issue_next:
            idx_issue(j + 2, b)
            idx_finish(j + 2, b)
        pltpu.make_async_copy(erows[b].at[pl.ds(0, NA)],
                              nh_sh.at[dstSA_v[b]], sem_snhA[b]).wait()
        pltpu.make_async_copy(erows[b].at[pl.ds(NA, NB)],
                              nh_sh.at[dstSB_v[b]], sem_snhB[b]).wait()
        if issue_next:
            egather_start(b)

    idx_issue(0, 0)
    idx_finish(0, 0)
    egather_start(0)
    ugatherA_start(0)
    ugatherB_start(0)
    idx_issue(1, 1)
    idx_finish(1, 1)
    egather_start(1)
    substep(0, 0, True, True, True)
    substep(1, 1, False, True, True)

    def pair_body(i, _):
        substep(2 * i, 0, False, True, True)
        substep(2 * i + 1, 1, False, True, True)
        return 0
    lax.fori_loop(1, (NCHUNK - 3) // 2, pair_body, 0)

    substep(NCHUNK - 3, 0, False, True, True)
    substep(NCHUNK - 2, 1, False, False, True)
    substep(NCHUNK - 1, 0, False, False, False)
    pltpu.make_async_copy(miniA, att_sh.at[arowA_v], sem_sa).wait()
    pltpu.make_async_copy(miniB, att_sh.at[arowB_v], sem_sb).wait()

    plsc.subcore_barrier()
    pltpu.sync_copy(nh_sh.at[pl.ds(s * ROWS_PER_TILE, ROWS_PER_TILE)],
                    nh_out.at[c, pl.ds(s * ROWS_PER_TILE, ROWS_PER_TILE)])

    @pl.when(s < ATILES)
    def _():
        pltpu.sync_copy(att_sh.at[pl.ds(s * ARPT, ARPT)],
                        att_out.at[c, pl.ds(s * ARPT, ARPT)])


@functools.partial(
    pl.kernel,
    out_type=[jax.ShapeDtypeStruct((NC, NP, D), jnp.float32),
              jax.ShapeDtypeStruct((NC, AR, D), jnp.float32)],
    mesh=plsc.VectorSubcoreMesh(core_axis_name="c", subcore_axis_name="s"),
    compiler_params=pltpu.CompilerParams(needs_layout_passes=False),
    scratch_types=[
        [pltpu.VMEM((3 * CH,), jnp.int32)] * 2,
        [pltpu.VMEM((CH,), jnp.int32)] * 2,
        [pltpu.VMEM((CH,), jnp.int32)] * 2,
        [pltpu.VMEM((NA,), jnp.int32)] * 2,
        [pltpu.VMEM((NB,), jnp.int32)] * 2,
        pltpu.VMEM((NA,), jnp.int32),
        pltpu.VMEM((NB,), jnp.int32),
        pltpu.VMEM((NA,), jnp.int32),
        pltpu.VMEM((NB,), jnp.int32),
        [pltpu.VMEM((CH, D), jnp.float32)] * 2,
        pltpu.VMEM((CH, D), jnp.float32),
        pltpu.VMEM((NA, D), jnp.float32),
        pltpu.VMEM((NB, D), jnp.float32),
        pltpu.VMEM_SHARED((NP, D), jnp.float32),
        pltpu.VMEM_SHARED((AR, D), jnp.float32),
        [pltpu.SemaphoreType.DMA] * 2,
        pltpu.SemaphoreType.DMA,
        pltpu.SemaphoreType.DMA,
        [pltpu.SemaphoreType.DMA] * 2,
        [pltpu.SemaphoreType.DMA] * 2,
        [pltpu.SemaphoreType.DMA] * 2,
        pltpu.SemaphoreType.DMA,
        pltpu.SemaphoreType.DMA,
    ],
)
def _sc_edge_pass(emb_hbm, u_hbm, sde_hbm, nh_out, att_out,
                  sde_v, gidx_v, arow_v, dstSA_v, dstSB_v,
                  laneA_v, laneB_v, arowA_v, arowB_v,
                  erows, urows, miniA, miniB, nh_sh, att_sh,
                  sem_ge, sem_guA, sem_guB, sem_snhA, sem_snhB, sem_idx, sem_sa,
                  sem_sb):
    _sc_body(emb_hbm, u_hbm, sde_hbm, nh_out, att_out,
             sde_v, gidx_v, arow_v, dstSA_v, dstSB_v,
             laneA_v, laneB_v, arowA_v, arowB_v,
             erows, urows, miniA, miniB, nh_sh, att_sh,
             sem_ge, sem_guA, sem_guB, sem_snhA, sem_snhB, sem_idx, sem_sa, sem_sb)


def _fin_body(emb_ref, nh_ref, att_ref, w1t_ref, b1_ref, w2t_ref, b2_ref,
              o_ref):
    att_sum = att_ref[0] + att_ref[1]
    nh = (nh_ref[0] + nh_ref[1]) / (att_sum + 1e-10)
    e = emb_ref[...]
    h1 = jnp.dot(e + nh, w1t_ref[...],
                 preferred_element_type=jnp.float32) + b1_ref[...]
    h2 = jnp.dot(e * nh, w2t_ref[...],
                 preferred_element_type=jnp.float32) + b2_ref[...]
    h1 = jnp.where(h1 > 0, h1, 0.01 * h1)
    h2 = jnp.where(h2 > 0, h2, 0.01 * h2)
    o = h1 + h2
    nrm = jnp.sqrt(jnp.sum(o * o, axis=1, keepdims=True))
    o_ref[:, :D] = e
    o_ref[:, D:] = o / jnp.maximum(nrm, 1e-12)


def _finalize(emb, nh, att, W1t, b1, W2t, b2):
    OUT = W1t.shape[1]
    return pl.pallas_call(
        _fin_body,
        grid=(N // FB,),
        in_specs=[
            pl.BlockSpec((FB, D), lambda i: (i, 0)),
            pl.BlockSpec((NC, FB, D), lambda i: (0, i, 0)),
            pl.BlockSpec((NC, FB, 1), lambda i: (0, i, 0)),
            pl.BlockSpec((D, OUT), lambda i: (0, 0)),
            pl.BlockSpec((1, OUT), lambda i: (0, 0)),
            pl.BlockSpec((D, OUT), lambda i: (0, 0)),
            pl.BlockSpec((1, OUT), lambda i: (0, 0)),
        ],
        out_specs=pl.BlockSpec((FB, D + OUT), lambda i: (i, 0)),
        out_shape=jax.ShapeDtypeStruct((N, D + OUT), jnp.float32),
    )(emb, nh, att, W1t, b1, W2t, b2)


def kernel(node_ids, edge_index, edge_type, emb_table, rel_embed, W_R,
           W1, b1, W2, b2):
    emb = jnp.take(emb_table, node_ids, axis=0)
    u = _precompute_u(emb, rel_embed, W_R)
    sde = (jnp.concatenate([edge_index, edge_type[None]], axis=0)
           .reshape(3, NWORK * NCHUNK, CH).transpose(1, 0, 2)
           .reshape(NWORK * NCHUNK, 3 * CH))
    nh, att = _sc_edge_pass(emb, u, sde)
    att_r = att.reshape(NC, NP, 1)
    return _finalize(emb, nh, att_r, W1.T, b1.reshape(1, -1),
                     W2.T, b2.reshape(1, -1))
